# baseline (device time: 72546 ns/iter reference)
import jax
import jax.numpy as jnp
from jax import lax
from jax.experimental import pallas as pl
from jax.experimental.pallas import tpu as pltpu

N_DEV = 4
N_LAYERS = 3


def kernel(x, Win0, Wout0, Win1, Wout1, Win2, Wout2):
    b, d = x.shape
    hs = Win0.shape[1]

    def body(x_ref, win0_ref, wout0_ref, win1_ref, wout1_ref, win2_ref,
             wout2_ref, out_ref, win_g, wout_g, send_sems, recv_sems):
        my = lax.axis_index("i")
        left = (my - 1) % N_DEV
        right = (my + 1) % N_DEV

        win_refs = [win0_ref, win1_ref, win2_ref]
        wout_refs = [wout0_ref, wout1_ref, wout2_ref]

        for l in range(N_LAYERS):
            win_g[my, l, :, :] = win_refs[l][:, :].astype(jnp.bfloat16)
            wout_g[my, l, :, :] = wout_refs[l][:, :].astype(jnp.bfloat16)

        barrier_sem = pltpu.get_barrier_semaphore()
        for nbr in (left, right):
            pl.semaphore_signal(
                barrier_sem, inc=1,
                device_id=(nbr,), device_id_type=pl.DeviceIdType.MESH,
            )
        pl.semaphore_wait(barrier_sem, 2)

        for h in range(N_DEV - 1):
            s_send = (my - h) % N_DEV
            s_recv = (my - h - 1) % N_DEV
            rdmas = []
            for l in range(N_LAYERS):
                for t, g in ((0, win_g), (1, wout_g)):
                    idx = h * (2 * N_LAYERS) + 2 * l + t
                    rdma = pltpu.make_async_remote_copy(
                        src_ref=g.at[s_send, l],
                        dst_ref=g.at[s_send, l],
                        send_sem=send_sems.at[idx],
                        recv_sem=recv_sems.at[idx],
                        device_id=(right,),
                        device_id_type=pl.DeviceIdType.MESH,
                    )
                    rdma.start()
                    rdmas.append(rdma)
            for l in range(N_LAYERS):
                for t, g in ((0, win_g), (1, wout_g)):
                    idx = h * (2 * N_LAYERS) + 2 * l + t
                    recv = pltpu.make_async_remote_copy(
                        src_ref=g.at[s_recv, l],
                        dst_ref=g.at[s_recv, l],
                        send_sem=send_sems.at[idx],
                        recv_sem=recv_sems.at[idx],
                        device_id=(left,),
                        device_id_type=pl.DeviceIdType.MESH,
                    )
                    recv.wait_recv()
            for rdma in rdmas:
                rdma.wait_send()

        x_cur = x_ref[:, :].astype(jnp.bfloat16)
        acc = jnp.zeros((b, d), jnp.float32)
        for l in range(N_LAYERS):
            acc = jnp.zeros((b, d), jnp.float32)
            for o in range(N_DEV):
                hchunk = jnp.dot(x_cur, win_g[o, l, :, :],
                                 preferred_element_type=jnp.float32)
                hchunk = jnp.maximum(hchunk, 0.0).astype(jnp.bfloat16)
                acc = acc + jnp.dot(hchunk, wout_g[o, l, :, :],
                                    preferred_element_type=jnp.float32)
            x_cur = acc.astype(jnp.bfloat16)
        out_ref[:, :] = acc

    n_sems = (N_DEV - 1) * 2 * N_LAYERS
    return pl.pallas_call(
        body,
        out_shape=jax.ShapeDtypeStruct((b, d), jnp.float32),
        in_specs=[pl.BlockSpec(memory_space=pltpu.VMEM)] * 7,
        out_specs=pl.BlockSpec(memory_space=pltpu.VMEM),
        scratch_shapes=[
            pltpu.VMEM((N_DEV, N_LAYERS, d, hs), jnp.bfloat16),
            pltpu.VMEM((N_DEV, N_LAYERS, hs, d), jnp.bfloat16),
            pltpu.SemaphoreType.DMA((n_sems,)),
            pltpu.SemaphoreType.DMA((n_sems,)),
        ],
        compiler_params=pltpu.CompilerParams(collective_id=0),
    )(x, Win0, Wout0, Win1, Wout1, Win2, Wout2)


# device time: 44024 ns/iter; 1.6479x vs baseline; 1.6479x over previous
import jax
import jax.numpy as jnp
from jax import lax
from jax.experimental import pallas as pl
from jax.experimental.pallas import tpu as pltpu

N_DEV = 4
N_LAYERS = 3


def kernel(x, Win0, Wout0, Win1, Wout1, Win2, Wout2):
    b, d = x.shape
    hs = Win0.shape[1]

    def body(x_ref, win0_ref, wout0_ref, win1_ref, wout1_ref, win2_ref,
             wout2_ref, out_ref, win_g, wout_g, send_sems, recv_sems):
        my = lax.axis_index("i")
        left = (my - 1) % N_DEV
        right = (my + 1) % N_DEV

        win_refs = [win0_ref, win1_ref, win2_ref]
        wout_refs = [wout0_ref, wout1_ref, wout2_ref]

        for l in range(N_LAYERS):
            win_g[my, l, :, :] = win_refs[l][:, :].astype(jnp.bfloat16)
            wout_g[my, l, :, :] = wout_refs[l][:, :].astype(jnp.bfloat16)

        barrier_sem = pltpu.get_barrier_semaphore()
        for nbr in (left, right):
            pl.semaphore_signal(
                barrier_sem, inc=1,
                device_id=(nbr,), device_id_type=pl.DeviceIdType.MESH,
            )
        pl.semaphore_wait(barrier_sem, 2)

        def mk(g, slot, idx, target):
            return pltpu.make_async_remote_copy(
                src_ref=g.at[slot],
                dst_ref=g.at[slot],
                send_sem=send_sems.at[idx],
                recv_sem=recv_sems.at[idx],
                device_id=(target,),
                device_id_type=pl.DeviceIdType.MESH,
            )

        sends = []
        for l in range(N_LAYERS):
            sends.append(mk(win_g, (my, l), 0 + l, right))
        for l in range(N_LAYERS):
            sends.append(mk(wout_g, (my, l), 3 + l, right))
        for l in range(N_LAYERS):
            sends.append(mk(wout_g, (my, l), 6 + l, left))
        for l in range(N_LAYERS):
            sends.append(mk(win_g, (my, l), 9 + l, left))
        for s in sends:
            s.start()

        for l in range(N_LAYERS):
            mk(win_g, (left, l), 0 + l, left).wait_recv()
        fwd_r = [mk(win_g, (left, l), 12 + l, right) for l in range(N_LAYERS)]
        for s in fwd_r:
            s.start()

        for l in range(N_LAYERS):
            mk(wout_g, (right, l), 6 + l, right).wait_recv()
        fwd_l = [mk(wout_g, (right, l), 15 + l, left) for l in range(N_LAYERS)]
        for s in fwd_l:
            s.start()

        opp = (my + 2) % N_DEV
        for l in range(N_LAYERS):
            mk(wout_g, (left, l), 3 + l, left).wait_recv()
            mk(win_g, (right, l), 9 + l, right).wait_recv()
            mk(win_g, (opp, l), 12 + l, left).wait_recv()
            mk(wout_g, (opp, l), 15 + l, right).wait_recv()
        for s in sends + fwd_r + fwd_l:
            s.wait_send()

        x_cur = x_ref[:, :].astype(jnp.bfloat16)
        acc = jnp.zeros((b, d), jnp.float32)
        for l in range(N_LAYERS):
            acc = jnp.zeros((b, d), jnp.float32)
            for o in range(N_DEV):
                hchunk = jnp.dot(x_cur, win_g[o, l, :, :],
                                 preferred_element_type=jnp.float32)
                hchunk = jnp.maximum(hchunk, 0.0).astype(jnp.bfloat16)
                acc = acc + jnp.dot(hchunk, wout_g[o, l, :, :],
                                    preferred_element_type=jnp.float32)
            x_cur = acc.astype(jnp.bfloat16)
        out_ref[:, :] = acc

    n_sems = 18
    return pl.pallas_call(
        body,
        out_shape=jax.ShapeDtypeStruct((b, d), jnp.float32),
        in_specs=[pl.BlockSpec(memory_space=pltpu.VMEM)] * 7,
        out_specs=pl.BlockSpec(memory_space=pltpu.VMEM),
        scratch_shapes=[
            pltpu.VMEM((N_DEV, N_LAYERS, d, hs), jnp.bfloat16),
            pltpu.VMEM((N_DEV, N_LAYERS, hs, d), jnp.bfloat16),
            pltpu.SemaphoreType.DMA((n_sems,)),
            pltpu.SemaphoreType.DMA((n_sems,)),
        ],
        compiler_params=pltpu.CompilerParams(collective_id=0),
    )(x, Win0, Wout0, Win1, Wout1, Win2, Wout2)


# device time: 38554 ns/iter; 1.8817x vs baseline; 1.1419x over previous
import jax
import jax.numpy as jnp
from jax import lax
from jax.experimental import pallas as pl
from jax.experimental.pallas import tpu as pltpu

N_DEV = 4
N_LAYERS = 3


def kernel(x, Win0, Wout0, Win1, Wout1, Win2, Wout2):
    b, d = x.shape
    hs = Win0.shape[1]

    x = x.astype(jnp.bfloat16)
    Win0 = Win0.astype(jnp.bfloat16)
    Wout0 = Wout0.astype(jnp.bfloat16)
    Win1 = Win1.astype(jnp.bfloat16)
    Wout1 = Wout1.astype(jnp.bfloat16)
    Win2 = Win2.astype(jnp.bfloat16)
    Wout2 = Wout2.astype(jnp.bfloat16)

    def body(x_hbm, win0_hbm, wout0_hbm, win1_hbm, wout1_hbm, win2_hbm,
             wout2_hbm, out_hbm, win_g, wout_g, x_st, out_st,
             send_sems, recv_sems, in_sems, out_sem):
        my = lax.axis_index("i")
        left = (my - 1) % N_DEV
        right = (my + 1) % N_DEV
        opp = (my + 2) % N_DEV

        win_hbm = [win0_hbm, win1_hbm, win2_hbm]
        wout_hbm = [wout0_hbm, wout1_hbm, wout2_hbm]

        barrier_sem = pltpu.get_barrier_semaphore()
        for nbr in (left, right):
            pl.semaphore_signal(
                barrier_sem, inc=1,
                device_id=(nbr,), device_id_type=pl.DeviceIdType.MESH,
            )

        in_dmas = []
        for l in range(N_LAYERS):
            cw = pltpu.make_async_copy(win_hbm[l], win_g.at[my, l],
                                       in_sems.at[2 * l])
            co = pltpu.make_async_copy(wout_hbm[l], wout_g.at[my, l],
                                       in_sems.at[2 * l + 1])
            cw.start()
            co.start()
            in_dmas.append((cw, co))
        cx = pltpu.make_async_copy(x_hbm, x_st, in_sems.at[6])
        cx.start()

        def mk(g, slot, idx, target):
            return pltpu.make_async_remote_copy(
                src_ref=g.at[slot],
                dst_ref=g.at[slot],
                send_sem=send_sems.at[idx],
                recv_sem=recv_sems.at[idx],
                device_id=(target,),
                device_id_type=pl.DeviceIdType.MESH,
            )

        def issue_hop1(l):
            cw, co = in_dmas[l]
            cw.wait()
            co.wait()
            group = [
                mk(win_g, (my, l), l * 6 + 0, right),
                mk(wout_g, (my, l), l * 6 + 2, left),
                mk(wout_g, (my, l), l * 6 + 1, right),
                mk(win_g, (my, l), l * 6 + 3, left),
            ]
            for s in group:
                s.start()
            return group

        pl.semaphore_wait(barrier_sem, 2)
        sends = issue_hop1(0)
        cx.wait()
        x_cur = x_st[:, :]

        acc = jnp.zeros((b, d), jnp.float32)
        for l in range(N_LAYERS):
            mk(win_g, (left, l), l * 6 + 0, left).wait_recv()
            fr = mk(win_g, (left, l), l * 6 + 4, right)
            fr.start()
            mk(wout_g, (right, l), l * 6 + 2, right).wait_recv()
            fl = mk(wout_g, (right, l), l * 6 + 5, left)
            fl.start()
            sends += [fr, fl]
            if l + 1 < N_LAYERS:
                sends += issue_hop1(l + 1)
            def contrib(o, acc):
                hchunk = jnp.dot(x_cur, win_g[o, l, :, :],
                                 preferred_element_type=jnp.float32)
                hchunk = jnp.maximum(hchunk, 0.0).astype(jnp.bfloat16)
                return acc + jnp.dot(hchunk, wout_g[o, l, :, :],
                                     preferred_element_type=jnp.float32)

            acc = jnp.zeros((b, d), jnp.float32)
            acc = contrib(my, acc)
            mk(wout_g, (left, l), l * 6 + 1, left).wait_recv()
            mk(win_g, (right, l), l * 6 + 3, right).wait_recv()
            acc = contrib(left, acc)
            acc = contrib(right, acc)
            mk(win_g, (opp, l), l * 6 + 4, left).wait_recv()
            mk(wout_g, (opp, l), l * 6 + 5, right).wait_recv()
            acc = contrib(opp, acc)
            x_cur = acc.astype(jnp.bfloat16)

        out_st[:, :] = acc
        cout = pltpu.make_async_copy(out_st, out_hbm, out_sem)
        cout.start()
        for s in sends:
            s.wait_send()
        cout.wait()

    n_sems = N_LAYERS * 6
    return pl.pallas_call(
        body,
        out_shape=jax.ShapeDtypeStruct((b, d), jnp.float32),
        in_specs=[pl.BlockSpec(memory_space=pl.ANY)] * 7,
        out_specs=pl.BlockSpec(memory_space=pl.ANY),
        scratch_shapes=[
            pltpu.VMEM((N_DEV, N_LAYERS, d, hs), jnp.bfloat16),
            pltpu.VMEM((N_DEV, N_LAYERS, hs, d), jnp.bfloat16),
            pltpu.VMEM((b, d), jnp.bfloat16),
            pltpu.VMEM((b, d), jnp.float32),
            pltpu.SemaphoreType.DMA((n_sems,)),
            pltpu.SemaphoreType.DMA((n_sems,)),
            pltpu.SemaphoreType.DMA((7,)),
            pltpu.SemaphoreType.DMA,
        ],
        compiler_params=pltpu.CompilerParams(collective_id=0),
    )(x, Win0, Wout0, Win1, Wout1, Win2, Wout2)
